# baseline (speedup 1.0000x reference)
import functools

import jax
import jax.numpy as jnp
from jax import lax
from jax.experimental import pallas as pl
from jax.experimental.pallas import tpu as pltpu
from jax.experimental.pallas import tpu_sc as plsc

_N = 1_000_000
_D = 32
_R = 800
_CH = _N // _R
_NW = 32
_NK = _CH // _NW
_EXTRA = _CH - _NK * _NW


def _compute_chunk(ub, vb, ob):
    def group(g, carry):
        rows = g * 16 + lax.iota(jnp.int32, 16)
        cols = jnp.zeros((16,), jnp.int32)
        acc = plsc.load_gather(ub, [rows, cols]) * plsc.load_gather(vb, [rows, cols])
        for j in range(1, _D):
            cols = jnp.full((16,), j, jnp.int32)
            acc = acc + plsc.load_gather(ub, [rows, cols]) * plsc.load_gather(vb, [rows, cols])
        ob[pl.ds(g * 16, 16)] = acc
        return carry

    lax.fori_loop(0, _R // 16, group, 0)


def _sc_rowdot(u_hbm, v_hbm, o_hbm, u0, u1, v0, v1, o0, o1,
               si0, si1, so0, so1):
    wid = lax.axis_index("s") * 2 + lax.axis_index("c")

    def start_in(c, ub, vb, sem):
        pltpu.async_copy(u_hbm.at[pl.ds(c * _R, _R)], ub, sem)
        pltpu.async_copy(v_hbm.at[pl.ds(c * _R, _R)], vb, sem)

    def wait_in(c, ub, vb, sem):
        pltpu.make_async_copy(u_hbm.at[pl.ds(c * _R, _R)], ub, sem).wait()
        pltpu.make_async_copy(v_hbm.at[pl.ds(c * _R, _R)], vb, sem).wait()

    def process(k, ub, vb, ob, sin, sout, ub_n, vb_n, sin_n):
        c = wid + k * _NW

        @pl.when(k + 1 < _NK)
        def _prefetch():
            start_in(c + _NW, ub_n, vb_n, sin_n)

        wait_in(c, ub, vb, sin)
        _compute_chunk(ub, vb, ob)

        @pl.when(k >= 2)
        def _drain_prev_out():
            prev = (c - 2 * _NW) * _R
            pltpu.make_async_copy(ob, o_hbm.at[pl.ds(prev, _R)], sout).wait()

        pltpu.async_copy(ob, o_hbm.at[pl.ds(c * _R, _R)], sout)

    start_in(wid, u0, v0, si0)

    def round_body(k, carry):
        @pl.when(k % 2 == 0)
        def _even():
            process(k, u0, v0, o0, si0, so0, u1, v1, si1)

        @pl.when(k % 2 == 1)
        def _odd():
            process(k, u1, v1, o1, si1, so1, u0, v0, si0)

        return carry

    lax.fori_loop(0, _NK, round_body, 0)

    pltpu.make_async_copy(o0, o_hbm.at[pl.ds(wid * _R, _R)], so0).wait()
    pltpu.make_async_copy(o1, o_hbm.at[pl.ds(wid * _R, _R)], so1).wait()

    @pl.when(wid < _EXTRA)
    def _tail():
        c = _NK * _NW + wid
        pltpu.sync_copy(u_hbm.at[pl.ds(c * _R, _R)], u0)
        pltpu.sync_copy(v_hbm.at[pl.ds(c * _R, _R)], v0)
        _compute_chunk(u0, v0, o0)
        pltpu.sync_copy(o0, o_hbm.at[pl.ds(c * _R, _R)])


def kernel(user_emb, items_emb):
    n, d = user_emb.shape
    mesh = plsc.VectorSubcoreMesh(core_axis_name="c", subcore_axis_name="s")
    run = functools.partial(
        pl.kernel,
        mesh=mesh,
        compiler_params=pltpu.CompilerParams(
            needs_layout_passes=False, use_tc_tiling_on_sc=False),
        out_type=jax.ShapeDtypeStruct((n,), jnp.float32),
        scratch_types=[
            pltpu.VMEM((_R, _D), jnp.float32),
            pltpu.VMEM((_R, _D), jnp.float32),
            pltpu.VMEM((_R, _D), jnp.float32),
            pltpu.VMEM((_R, _D), jnp.float32),
            pltpu.VMEM((_R,), jnp.float32),
            pltpu.VMEM((_R,), jnp.float32),
            pltpu.SemaphoreType.DMA,
            pltpu.SemaphoreType.DMA,
            pltpu.SemaphoreType.DMA,
            pltpu.SemaphoreType.DMA,
        ],
    )(_sc_rowdot)
    return run(user_emb, items_emb)


# speedup vs baseline: 1.8941x; 1.8941x over previous
import functools

import jax
import jax.numpy as jnp
from jax import lax
from jax.experimental import pallas as pl
from jax.experimental.pallas import tpu as pltpu
from jax.experimental.pallas import tpu_sc as plsc

_N = 1_000_000
_D = 32
_R = 800
_CH = _N // _R
_NW = 32
_NK = _CH // _NW
_EXTRA = _CH - _NK * _NW


def _compute_chunk(ub, vb, ob):
    _LANE = lax.iota(jnp.int32, 16)

    def group(g, carry):
        base = g * 16
        acc = jnp.zeros((16,), jnp.float32)
        for i in range(16):
            r = base + i
            s = (ub[r, pl.ds(0, 16)] * vb[r, pl.ds(0, 16)]
                 + ub[r, pl.ds(16, 16)] * vb[r, pl.ds(16, 16)])
            acc = jnp.where(_LANE == i, jnp.sum(s), acc)
        ob[pl.ds(base, 16)] = acc
        return carry

    lax.fori_loop(0, _R // 16, group, 0)


def _sc_rowdot(u_hbm, v_hbm, o_hbm, u0, u1, v0, v1, o0, o1,
               si0, si1, so0, so1):
    wid = lax.axis_index("s") * 2 + lax.axis_index("c")

    def start_in(c, ub, vb, sem):
        pltpu.async_copy(u_hbm.at[pl.ds(c * _R, _R)], ub, sem)
        pltpu.async_copy(v_hbm.at[pl.ds(c * _R, _R)], vb, sem)

    def wait_in(c, ub, vb, sem):
        pltpu.make_async_copy(u_hbm.at[pl.ds(c * _R, _R)], ub, sem).wait()
        pltpu.make_async_copy(v_hbm.at[pl.ds(c * _R, _R)], vb, sem).wait()

    def process(k, ub, vb, ob, sin, sout, ub_n, vb_n, sin_n):
        c = wid + k * _NW

        @pl.when(k + 1 < _NK)
        def _prefetch():
            start_in(c + _NW, ub_n, vb_n, sin_n)

        wait_in(c, ub, vb, sin)
        _compute_chunk(ub, vb, ob)

        @pl.when(k >= 2)
        def _drain_prev_out():
            prev = (c - 2 * _NW) * _R
            pltpu.make_async_copy(ob, o_hbm.at[pl.ds(prev, _R)], sout).wait()

        pltpu.async_copy(ob, o_hbm.at[pl.ds(c * _R, _R)], sout)

    start_in(wid, u0, v0, si0)

    def round_body(k, carry):
        @pl.when(k % 2 == 0)
        def _even():
            process(k, u0, v0, o0, si0, so0, u1, v1, si1)

        @pl.when(k % 2 == 1)
        def _odd():
            process(k, u1, v1, o1, si1, so1, u0, v0, si0)

        return carry

    lax.fori_loop(0, _NK, round_body, 0)

    pltpu.make_async_copy(o0, o_hbm.at[pl.ds(wid * _R, _R)], so0).wait()
    pltpu.make_async_copy(o1, o_hbm.at[pl.ds(wid * _R, _R)], so1).wait()

    @pl.when(wid < _EXTRA)
    def _tail():
        c = _NK * _NW + wid
        pltpu.sync_copy(u_hbm.at[pl.ds(c * _R, _R)], u0)
        pltpu.sync_copy(v_hbm.at[pl.ds(c * _R, _R)], v0)
        _compute_chunk(u0, v0, o0)
        pltpu.sync_copy(o0, o_hbm.at[pl.ds(c * _R, _R)])


def kernel(user_emb, items_emb):
    n, d = user_emb.shape
    mesh = plsc.VectorSubcoreMesh(core_axis_name="c", subcore_axis_name="s")
    run = functools.partial(
        pl.kernel,
        mesh=mesh,
        compiler_params=pltpu.CompilerParams(
            needs_layout_passes=False, use_tc_tiling_on_sc=False),
        out_type=jax.ShapeDtypeStruct((n,), jnp.float32),
        scratch_types=[
            pltpu.VMEM((_R, _D), jnp.float32),
            pltpu.VMEM((_R, _D), jnp.float32),
            pltpu.VMEM((_R, _D), jnp.float32),
            pltpu.VMEM((_R, _D), jnp.float32),
            pltpu.VMEM((_R,), jnp.float32),
            pltpu.VMEM((_R,), jnp.float32),
            pltpu.SemaphoreType.DMA,
            pltpu.SemaphoreType.DMA,
            pltpu.SemaphoreType.DMA,
            pltpu.SemaphoreType.DMA,
        ],
    )(_sc_rowdot)
    return run(user_emb, items_emb)


# speedup vs baseline: 24.2202x; 12.7871x over previous
import jax
import jax.numpy as jnp
from jax.experimental import pallas as pl

_BLOCK = 65536


def _rowdot_body(u_ref, v_ref, o_ref):
    o_ref[...] = jnp.sum(u_ref[...] * v_ref[...], axis=0)


def kernel(user_emb, items_emb):
    n, d = user_emb.shape
    ut = user_emb.T
    vt = items_emb.T
    return pl.pallas_call(
        _rowdot_body,
        grid=(pl.cdiv(n, _BLOCK),),
        in_specs=[
            pl.BlockSpec((d, _BLOCK), lambda i: (0, i)),
            pl.BlockSpec((d, _BLOCK), lambda i: (0, i)),
        ],
        out_specs=pl.BlockSpec((_BLOCK,), lambda i: (i,)),
        out_shape=jax.ShapeDtypeStruct((n,), jnp.float32),
    )(ut, vt)
